# Initial kernel scaffold; baseline (speedup 1.0000x reference)
import jax
import jax.numpy as jnp
from jax.experimental import pallas as pl


def kernel(x, edge_index, W1, b1, W2, b2):
    raise NotImplementedError("write your pallas kernel here")


# speedup vs baseline: 8.5673x; 8.5673x over previous
import functools

import jax
import jax.numpy as jnp
from jax import lax
from jax.experimental import pallas as pl
from jax.experimental.pallas import tpu as pltpu
from jax.experimental.pallas import tpu_sc as plsc

N = 10000
D = 256
HF = 128
E = 160000
NC = 2
NS = 16
CH = 128
EPT = 10240
E_PAD = EPT * NS
NCH_G = EPT // CH
EPT_DEG = E_PAD // (NC * NS)
NCH_DEG = EPT_DEG // CH
N_ACC = 10240
ROWS_PT = N_ACC // NS
ROWS_LAST = N - (NS - 1) * ROWS_PT
NBUF = 2
NSTG = 2
CHPS = NCH_G // NSTG
MB = 512
GRID_M = N_ACC // MB

_MESH = dict(core_axis_name="c", subcore_axis_name="s",
             num_cores=NC, num_subcores=NS)


@functools.partial(
    pl.kernel,
    out_type=jax.ShapeDtypeStruct((NC, N_ACC), jnp.float32),
    mesh=plsc.VectorSubcoreMesh(**_MESH),
    scratch_types=[
        pltpu.VMEM((NCH_DEG, CH), jnp.int32),
        pltpu.VMEM((CH,), jnp.float32),
        pltpu.VMEM((ROWS_PT,), jnp.float32),
        pltpu.VMEM_SHARED((N_ACC,), jnp.float32),
    ],
)
def _deg_kernel(dst_hbm, out_hbm, idx_v, ones_v, zero_v, acc):
    c = lax.axis_index("c")
    s = lax.axis_index("s")
    w = c * NS + s
    for i in range(CH // 16):
        ones_v[pl.ds(i * 16, 16)] = jnp.ones((16,), jnp.float32)
    for i in range(ROWS_PT // 16):
        zero_v[pl.ds(i * 16, 16)] = jnp.zeros((16,), jnp.float32)
    pltpu.sync_copy(dst_hbm.at[pl.ds(w * NCH_DEG, NCH_DEG)], idx_v)
    pltpu.sync_copy(zero_v, acc.at[pl.ds(s * ROWS_PT, ROWS_PT)])
    plsc.subcore_barrier()

    def body(j):
        pltpu.sync_copy(ones_v, acc.at[idx_v.at[j]], add=True)

    pl.loop(0, NCH_DEG)(body)
    plsc.subcore_barrier()
    pltpu.sync_copy(acc.at[pl.ds(s * ROWS_PT, ROWS_PT)],
                    out_hbm.at[c, pl.ds(s * ROWS_PT, ROWS_PT)])


@functools.partial(
    pl.kernel,
    out_type=(jax.ShapeDtypeStruct((N, HF), jnp.float32),
              jax.ShapeDtypeStruct((N, HF), jnp.float32)),
    mesh=plsc.VectorSubcoreMesh(**_MESH),
    scratch_types=[
        pltpu.VMEM((CHPS, CH), jnp.int32),
        pltpu.VMEM((CHPS, CH), jnp.int32),
        pltpu.VMEM_SHARED((N_ACC, HF), jnp.float32),
    ] + [pltpu.VMEM((CH, HF), jnp.float32) for _ in range(NBUF)]
      + [pltpu.SemaphoreType.DMA for _ in range(NBUF)],
)
def _agg_kernel(h0, h1, src_hbm, dst_hbm, o0, o1,
                idx_s, idx_d, acc, r0, r1, m0, m1):
    c = lax.axis_index("c")
    s = lax.axis_index("s")
    rows = (r0, r1)
    sems = (m0, m1)
    base_r = s * ROWS_PT

    def half(h_ref, o_ref):
        @pl.when(s < NS - 1)
        def _():
            pltpu.sync_copy(h_ref.at[pl.ds(base_r, ROWS_PT)],
                            acc.at[pl.ds(base_r, ROWS_PT)])

        @pl.when(s == NS - 1)
        def _():
            pltpu.sync_copy(h_ref.at[pl.ds(base_r, ROWS_LAST)],
                            acc.at[pl.ds(base_r, ROWS_LAST)])

        plsc.subcore_barrier()

        for g in range(NSTG):
            ebase = s * NCH_G + g * CHPS
            pltpu.sync_copy(src_hbm.at[pl.ds(ebase, CHPS)], idx_s)
            pltpu.sync_copy(dst_hbm.at[pl.ds(ebase, CHPS)], idx_d)

            for b in range(NBUF):
                pltpu.async_copy(h_ref.at[idx_s.at[b]], rows[b], sems[b])

            def body(j):
                for b in range(NBUF):
                    jj = j + b
                    pltpu.make_async_copy(h_ref.at[idx_s.at[jj]], rows[b],
                                          sems[b]).wait()
                    pltpu.sync_copy(rows[b], acc.at[idx_d.at[jj]], add=True)

                    @pl.when(jj + NBUF < CHPS)
                    def _():
                        pltpu.async_copy(h_ref.at[idx_s.at[jj + NBUF]],
                                         rows[b], sems[b])

            pl.loop(0, CHPS, step=NBUF)(body)
        plsc.subcore_barrier()

        @pl.when(s < NS - 1)
        def _():
            pltpu.sync_copy(acc.at[pl.ds(base_r, ROWS_PT)],
                            o_ref.at[pl.ds(base_r, ROWS_PT)])

        @pl.when(s == NS - 1)
        def _():
            pltpu.sync_copy(acc.at[pl.ds(base_r, ROWS_LAST)],
                            o_ref.at[pl.ds(base_r, ROWS_LAST)])

    @pl.when(c == 0)
    def _():
        half(h0, o0)

    @pl.when(c == 1)
    def _():
        half(h1, o1)


def _m1_body(x_ref, w_ref, d0_ref, d1_ref, h0_ref, h1_ref, dinv_ref):
    deg = d0_ref[...] + d1_ref[...] + 1.0
    dinv = lax.rsqrt(deg)
    h = jnp.dot(x_ref[...], w_ref[...],
                preferred_element_type=jnp.float32) * dinv
    h0_ref[...] = h[:, :HF]
    h1_ref[...] = h[:, HF:]
    dinv_ref[...] = dinv


def _m1(x, W1, d0, d1):
    return pl.pallas_call(
        _m1_body,
        grid=(GRID_M,),
        in_specs=[
            pl.BlockSpec((MB, D), lambda i: (i, 0)),
            pl.BlockSpec((D, D), lambda i: (0, 0)),
            pl.BlockSpec((MB, 1), lambda i: (i, 0)),
            pl.BlockSpec((MB, 1), lambda i: (i, 0)),
        ],
        out_specs=[
            pl.BlockSpec((MB, HF), lambda i: (i, 0)),
            pl.BlockSpec((MB, HF), lambda i: (i, 0)),
            pl.BlockSpec((MB, 1), lambda i: (i, 0)),
        ],
        out_shape=[
            jax.ShapeDtypeStruct((N, HF), jnp.float32),
            jax.ShapeDtypeStruct((N, HF), jnp.float32),
            jax.ShapeDtypeStruct((N_ACC, 1), jnp.float32),
        ],
    )(x, W1, d0, d1)


def _m2_body(o0_ref, o1_ref, dinv_ref, b1_ref, w2_ref, g0_ref, g1_ref):
    dinv = dinv_ref[...]
    t = jnp.concatenate([o0_ref[...], o1_ref[...]], axis=1) * dinv + b1_ref[...]
    t = jnp.maximum(t, 0.0)
    g = jnp.dot(t, w2_ref[...], preferred_element_type=jnp.float32) * dinv
    g0_ref[...] = g[:, :HF]
    g1_ref[...] = g[:, HF:]


def _m2(o0, o1, dinv, b1r, W2):
    return pl.pallas_call(
        _m2_body,
        grid=(GRID_M,),
        in_specs=[
            pl.BlockSpec((MB, HF), lambda i: (i, 0)),
            pl.BlockSpec((MB, HF), lambda i: (i, 0)),
            pl.BlockSpec((MB, 1), lambda i: (i, 0)),
            pl.BlockSpec((1, D), lambda i: (0, 0)),
            pl.BlockSpec((D, D), lambda i: (0, 0)),
        ],
        out_specs=[
            pl.BlockSpec((MB, HF), lambda i: (i, 0)),
            pl.BlockSpec((MB, HF), lambda i: (i, 0)),
        ],
        out_shape=[
            jax.ShapeDtypeStruct((N, HF), jnp.float32),
            jax.ShapeDtypeStruct((N, HF), jnp.float32),
        ],
    )(o0, o1, dinv, b1r, W2)


def _m3_body(p0_ref, p1_ref, dinv_ref, b2_ref, out_ref):
    out_ref[...] = (jnp.concatenate([p0_ref[...], p1_ref[...]], axis=1)
                    * dinv_ref[...] + b2_ref[...])


def _m3(p0, p1, dinv, b2r):
    return pl.pallas_call(
        _m3_body,
        grid=(GRID_M,),
        in_specs=[
            pl.BlockSpec((MB, HF), lambda i: (i, 0)),
            pl.BlockSpec((MB, HF), lambda i: (i, 0)),
            pl.BlockSpec((MB, 1), lambda i: (i, 0)),
            pl.BlockSpec((1, D), lambda i: (0, 0)),
        ],
        out_specs=pl.BlockSpec((MB, D), lambda i: (i, 0)),
        out_shape=jax.ShapeDtypeStruct((N, D), jnp.float32),
    )(p0, p1, dinv, b2r)


def kernel(x, edge_index, W1, b1, W2, b2):
    src = edge_index[0]
    dst = edge_index[1]
    pad = E_PAD - E
    src_r = jnp.concatenate(
        [src, jnp.zeros((pad,), src.dtype)]).reshape(E_PAD // CH, CH)
    dst_r = jnp.concatenate(
        [dst, jnp.full((pad,), N, dst.dtype)]).reshape(E_PAD // CH, CH)

    degp = _deg_kernel(dst_r)
    d0 = degp[0].reshape(N_ACC, 1)
    d1 = degp[1].reshape(N_ACC, 1)

    h0, h1, dinv = _m1(x, W1, d0, d1)
    o0, o1 = _agg_kernel(h0, h1, src_r, dst_r)
    g0, g1 = _m2(o0, o1, dinv, b1.reshape(1, D), W2)
    p0, p1 = _agg_kernel(g0, g1, src_r, dst_r)
    return _m3(p0, p1, dinv, b2.reshape(1, D))
